# baseline (speedup 1.0000x reference)
import functools

import jax
import jax.numpy as jnp
from jax import lax
from jax.experimental import pallas as pl
from jax.experimental.pallas import tpu as pltpu
from jax.experimental.pallas import tpu_sc as plsc

N = 10000
E = 160000
HID = 256
NH = HID // 2
CH = 128
NC = 2
NS = 16
NPAD = N + 8
CPS = 80
NHALF = 2
HCH = CPS // NHALF
EPAD = NS * CPS * CH
DEGC = CPS // NC
ROWS_PER_SUB = 624
ROWS_TAIL = NPAD - NS * ROWS_PER_SUB

RB = 2000
GRID = N // RB


def _sc_mesh():
    return plsc.VectorSubcoreMesh(
        core_axis_name="c", subcore_axis_name="s", num_cores=NC, num_subcores=NS)


def _zero_acc(z_hbm, acc_sh, s):
    rows = pl.ds(s * ROWS_PER_SUB, ROWS_PER_SUB)
    pltpu.sync_copy(z_hbm.at[rows], acc_sh.at[rows])

    @pl.when(s == NS - 1)
    def _():
        tail = pl.ds(NS * ROWS_PER_SUB, ROWS_TAIL)
        pltpu.sync_copy(z_hbm.at[tail], acc_sh.at[tail])


def _writeout(acc_sh, out_hbm, c, s):
    rows = pl.ds(s * ROWS_PER_SUB, ROWS_PER_SUB)
    pltpu.sync_copy(acc_sh.at[rows], out_hbm.at[c, rows])

    @pl.when(s == NS - 1)
    def _():
        tail = pl.ds(NS * ROWS_PER_SUB, ROWS_TAIL)
        pltpu.sync_copy(acc_sh.at[tail], out_hbm.at[c, tail])


def _sc_degree(src3dw, zeros_nh, ones_nh):
    @functools.partial(
        pl.kernel,
        out_type=jax.ShapeDtypeStruct((NC, NPAD, NH), jnp.float32),
        mesh=_sc_mesh(),
        scratch_types=[
            pltpu.VMEM((DEGC, CH), jnp.int32),
            pltpu.VMEM((CH, NH), jnp.float32),
            pltpu.VMEM_SHARED((NPAD, NH), jnp.float32),
            pltpu.SemaphoreType.DMA,
        ],
    )
    def deg_kernel(src_hbm, z_hbm, ones_hbm, out_hbm, si_d, ones_v, acc_sh, sem):
        c = lax.axis_index("c")
        s = lax.axis_index("s")
        _zero_acc(z_hbm, acc_sh, s)
        pltpu.sync_copy(src_hbm.at[s, pl.ds(c * DEGC, DEGC)], si_d)
        pltpu.sync_copy(ones_hbm, ones_v)
        plsc.subcore_barrier()

        @pl.loop(0, DEGC)
        def _(k):
            pltpu.async_copy(ones_v, acc_sh.at[si_d.at[k]], sem, add=True)

        @pl.loop(0, DEGC)
        def _(k):
            pltpu.make_async_copy(ones_v, acc_sh.at[si_d.at[0]], sem).wait()

        plsc.subcore_barrier()
        _writeout(acc_sh, out_hbm, c, s)

    return deg_kernel(src3dw, zeros_nh, ones_nh)


def _sc_segsum(hs0p, hs1p, src3d, dst3d, zeros_nh):
    @functools.partial(
        pl.kernel,
        out_type=jax.ShapeDtypeStruct((NC, NPAD, NH), jnp.float32),
        mesh=_sc_mesh(),
        scratch_types=[
            pltpu.VMEM((HCH, CH), jnp.int32),
            pltpu.VMEM((HCH, CH), jnp.int32),
            pltpu.VMEM((CH, NH), jnp.float32),
            pltpu.VMEM_SHARED((NPAD, NH), jnp.float32),
        ],
    )
    def seg_kernel(h0_hbm, h1_hbm, src_hbm, dst_hbm, z_hbm, out_hbm,
                   si_v, di_v, buf0, acc_sh):
        c = lax.axis_index("c")
        s = lax.axis_index("s")
        _zero_acc(z_hbm, acc_sh, s)
        plsc.subcore_barrier()

        @pl.loop(0, NHALF)
        def _(hh):
            off = pl.multiple_of(hh * HCH, 8)
            pltpu.sync_copy(src_hbm.at[s, pl.ds(off, HCH)], si_v)
            pltpu.sync_copy(dst_hbm.at[s, pl.ds(off, HCH)], di_v)

            @pl.loop(0, HCH)
            def _(j):
                @pl.when(c == 0)
                def _():
                    pltpu.sync_copy(h0_hbm.at[si_v.at[j]], buf0)

                @pl.when(c == 1)
                def _():
                    pltpu.sync_copy(h1_hbm.at[si_v.at[j]], buf0)

                pltpu.sync_copy(buf0, acc_sh.at[di_v.at[j]], add=True)

        plsc.subcore_barrier()
        _writeout(acc_sh, out_hbm, c, s)

    return seg_kernel(hs0p, hs1p, src3d, dst3d, zeros_nh)


def _tc_step0(x0, wx, bc):
    def body(x_ref, wx_ref, b_ref, h_ref, c_ref):
        z = x_ref[...] * wx_ref[...] + b_ref[...]
        i = jax.nn.sigmoid(z[:, 0:HID])
        t = jnp.tanh(z[:, 2 * HID:3 * HID])
        o = jax.nn.sigmoid(z[:, 3 * HID:4 * HID])
        cc = i * t
        c_ref[...] = cc
        h_ref[...] = o * jnp.tanh(cc)

    return pl.pallas_call(
        body,
        grid=(GRID,),
        in_specs=[
            pl.BlockSpec((RB, 1), lambda i: (i, 0)),
            pl.BlockSpec((1, 4 * HID), lambda i: (0, 0)),
            pl.BlockSpec((1, 4 * HID), lambda i: (0, 0)),
        ],
        out_specs=[pl.BlockSpec((RB, HID), lambda i: (i, 0))] * 2,
        out_shape=[jax.ShapeDtypeStruct((N, HID), jnp.float32)] * 2,
    )(x0, wx, bc)


def _tc_prep(h, degp):
    def body(h_ref, dp_ref, dinv_ref, h0_ref, h1_ref):
        deg = dp_ref[0][:, 0:1] + dp_ref[1][:, 0:1]
        dinv = jnp.where(deg > 0, 1.0 / jnp.sqrt(jnp.maximum(deg, 1e-12)), 0.0)
        dinv_ref[...] = dinv
        hs = h_ref[...] * dinv
        h0_ref[...] = hs[:, 0:NH]
        h1_ref[...] = hs[:, NH:HID]

    return pl.pallas_call(
        body,
        grid=(GRID,),
        in_specs=[
            pl.BlockSpec((RB, HID), lambda i: (i, 0)),
            pl.BlockSpec((NC, RB, NH), lambda i: (0, i, 0)),
        ],
        out_specs=[
            pl.BlockSpec((RB, 1), lambda i: (i, 0)),
            pl.BlockSpec((RB, NH), lambda i: (i, 0)),
            pl.BlockSpec((RB, NH), lambda i: (i, 0)),
        ],
        out_shape=[
            jax.ShapeDtypeStruct((N, 1), jnp.float32),
            jax.ShapeDtypeStruct((NPAD, NH), jnp.float32),
            jax.ShapeDtypeStruct((NPAD, NH), jnp.float32),
        ],
    )(h, degp)


def _tc_step_a(xt, h, w0, wx, bc):
    def body(x_ref, h_ref, w0_ref, wx_ref, b_ref, p_ref):
        p = jnp.dot(h_ref[...], w0_ref[...], preferred_element_type=jnp.float32)
        p_ref[...] = p + x_ref[...] * wx_ref[...] + b_ref[...]

    return pl.pallas_call(
        body,
        grid=(GRID,),
        in_specs=[
            pl.BlockSpec((RB, 1), lambda i: (i, 0)),
            pl.BlockSpec((RB, HID), lambda i: (i, 0)),
            pl.BlockSpec((HID, 4 * HID), lambda i: (0, 0)),
            pl.BlockSpec((1, 4 * HID), lambda i: (0, 0)),
            pl.BlockSpec((1, 4 * HID), lambda i: (0, 0)),
        ],
        out_specs=[pl.BlockSpec((RB, 4 * HID), lambda i: (i, 0))],
        out_shape=[jax.ShapeDtypeStruct((N, 4 * HID), jnp.float32)],
    )(xt, h, w0, wx, bc)[0]


def _tc_step_b(p, c, tx, dinv, w1a, w1b):
    def body(p_ref, c_ref, t_ref, dv_ref, w1a_ref, w1b_ref,
             hn_ref, cn_ref, h0_ref, h1_ref):
        dv = dv_ref[...]
        nd = -dv
        z = p_ref[...]
        z = z + jnp.dot(t_ref[0] * nd, w1a_ref[...],
                        preferred_element_type=jnp.float32)
        z = z + jnp.dot(t_ref[1] * nd, w1b_ref[...],
                        preferred_element_type=jnp.float32)
        i = jax.nn.sigmoid(z[:, 0:HID])
        f = jax.nn.sigmoid(z[:, HID:2 * HID])
        t = jnp.tanh(z[:, 2 * HID:3 * HID])
        o = jax.nn.sigmoid(z[:, 3 * HID:4 * HID])
        cn = f * c_ref[...] + i * t
        hn = o * jnp.tanh(cn)
        hn_ref[...] = hn
        cn_ref[...] = cn
        hs = hn * dv
        h0_ref[...] = hs[:, 0:NH]
        h1_ref[...] = hs[:, NH:HID]

    return pl.pallas_call(
        body,
        grid=(GRID,),
        in_specs=[
            pl.BlockSpec((RB, 4 * HID), lambda i: (i, 0)),
            pl.BlockSpec((RB, HID), lambda i: (i, 0)),
            pl.BlockSpec((NC, RB, NH), lambda i: (0, i, 0)),
            pl.BlockSpec((RB, 1), lambda i: (i, 0)),
            pl.BlockSpec((NH, 4 * HID), lambda i: (0, 0)),
            pl.BlockSpec((NH, 4 * HID), lambda i: (0, 0)),
        ],
        out_specs=[
            pl.BlockSpec((RB, HID), lambda i: (i, 0)),
            pl.BlockSpec((RB, HID), lambda i: (i, 0)),
            pl.BlockSpec((RB, NH), lambda i: (i, 0)),
            pl.BlockSpec((RB, NH), lambda i: (i, 0)),
        ],
        out_shape=[
            jax.ShapeDtypeStruct((N, HID), jnp.float32),
            jax.ShapeDtypeStruct((N, HID), jnp.float32),
            jax.ShapeDtypeStruct((NPAD, NH), jnp.float32),
            jax.ShapeDtypeStruct((NPAD, NH), jnp.float32),
        ],
    )(p, c, tx, dinv, w1a, w1b)


def kernel(x_seq, edge_index, W_i, b_i, Wch_i, bch_i, W_f, b_f, Wch_f, bch_f,
           W_c, b_c, Wch_c, bch_c, W_o, b_o, Wch_o, bch_o):
    pad = jnp.full((EPAD - E,), N, jnp.int32)
    bsrc = jnp.concatenate([edge_index[0], pad]).reshape(NS, CPS, CH)
    bdst = jnp.concatenate([edge_index[1], pad]).reshape(NS, CPS, CH)

    w0 = jnp.concatenate([Wch_i[0], Wch_f[0], Wch_c[0], Wch_o[0]], axis=1)
    w1 = jnp.concatenate([Wch_i[1], Wch_f[1], Wch_c[1], Wch_o[1]], axis=1)
    w1a = w1[0:NH]
    w1b = w1[NH:HID]
    wx = jnp.concatenate([W_i, W_f, W_c, W_o], axis=1)
    bc = jnp.concatenate([
        b_i + bch_i[None, :], b_f + bch_f[None, :],
        b_c + bch_c[None, :], b_o + bch_o[None, :]], axis=1)

    zeros_nh = jnp.zeros((NPAD, NH), jnp.float32)
    ones_nh = jnp.ones((CH, NH), jnp.float32)

    degp = _sc_degree(bsrc, zeros_nh, ones_nh)
    h, cst = _tc_step0(x_seq[0], wx, bc)
    dinv, hs0, hs1 = _tc_prep(h, degp)

    for t in range(1, 3):
        tx = _sc_segsum(hs0, hs1, bsrc, bdst, zeros_nh)
        p = _tc_step_a(x_seq[t], h, w0, wx, bc)
        h, cst, hs0, hs1 = _tc_step_b(p, cst, tx, dinv, w1a, w1b)

    return h


# speedup vs baseline: 1.4389x; 1.4389x over previous
import functools

import jax
import jax.numpy as jnp
from jax import lax
from jax.experimental import pallas as pl
from jax.experimental.pallas import tpu as pltpu
from jax.experimental.pallas import tpu_sc as plsc

N = 10000
E = 160000
HID = 256
NH = HID // 2
CH = 128
NCHUNK = E // CH
NC = 2
NS = 16
ROWS_PER_SUB = 624
ROWS_TAIL = N - NS * ROWS_PER_SUB

DEG_PER = NCHUNK // (NC * NS)
DEG_EXTRA = NCHUNK - DEG_PER * NC * NS
SEG_PER = NCHUNK // NS
SEG_EXTRA = NCHUNK - SEG_PER * NS

RB = 2000
GRID = N // RB

def _sc_mesh():
    return plsc.VectorSubcoreMesh(
        core_axis_name="c", subcore_axis_name="s", num_cores=NC, num_subcores=NS)


def _sc_degree(src, zeros_nh, ones_nh):
    @functools.partial(
        pl.kernel,
        out_type=[jax.ShapeDtypeStruct((N, NH), jnp.float32)] * 2,
        mesh=_sc_mesh(),
        scratch_types=[
            pltpu.VMEM((CH,), jnp.int32),
            pltpu.VMEM((CH, NH), jnp.float32),
            pltpu.VMEM_SHARED((N, NH), jnp.float32),
        ],
    )
    def deg_kernel(src_hbm, z_hbm, ones_hbm, outa_hbm, outb_hbm,
                   idx_v, ones_v, acc_sh):
        c = lax.axis_index("c")
        s = lax.axis_index("s")
        w = c * NS + s
        rows = pl.ds(s * ROWS_PER_SUB, ROWS_PER_SUB)
        tail = pl.ds(NS * ROWS_PER_SUB, ROWS_TAIL)
        pltpu.sync_copy(z_hbm.at[rows], acc_sh.at[rows])

        @pl.when(s == NS - 1)
        def _():
            pltpu.sync_copy(z_hbm.at[tail], acc_sh.at[tail])

        pltpu.sync_copy(ones_hbm, ones_v)
        plsc.subcore_barrier()

        def do_chunk(k):
            off = pl.multiple_of(k * CH, CH)
            pltpu.sync_copy(src_hbm.at[pl.ds(off, CH)], idx_v)
            pltpu.sync_copy(ones_v, acc_sh.at[idx_v], add=True)

        @pl.loop(0, DEG_PER)
        def _(j):
            do_chunk(w * DEG_PER + j)

        @pl.when(w < DEG_EXTRA)
        def _():
            do_chunk(NC * NS * DEG_PER + w)

        plsc.subcore_barrier()

        @pl.when(c == 0)
        def _():
            pltpu.sync_copy(acc_sh.at[rows], outa_hbm.at[rows])

            @pl.when(s == NS - 1)
            def _():
                pltpu.sync_copy(acc_sh.at[tail], outa_hbm.at[tail])

        @pl.when(c == 1)
        def _():
            pltpu.sync_copy(acc_sh.at[rows], outb_hbm.at[rows])

            @pl.when(s == NS - 1)
            def _():
                pltpu.sync_copy(acc_sh.at[tail], outb_hbm.at[tail])

    return deg_kernel(src, zeros_nh, ones_nh)


def _sc_segsum(hs0, hs1, sd, zeros_nh):
    @functools.partial(
        pl.kernel,
        out_type=[jax.ShapeDtypeStruct((N, NH), jnp.float32)] * 2,
        mesh=_sc_mesh(),
        scratch_types=[
            pltpu.VMEM((2, CH), jnp.int32),
            pltpu.VMEM((CH, NH), jnp.float32),
            pltpu.VMEM_SHARED((N, NH), jnp.float32),
        ],
    )
    def seg_kernel(h0_hbm, h1_hbm, sd_hbm, z_hbm, t0_hbm, t1_hbm,
                   sd_v, rows_v, acc_sh):
        c = lax.axis_index("c")
        s = lax.axis_index("s")
        rows = pl.ds(s * ROWS_PER_SUB, ROWS_PER_SUB)
        tail = pl.ds(NS * ROWS_PER_SUB, ROWS_TAIL)
        pltpu.sync_copy(z_hbm.at[rows], acc_sh.at[rows])

        @pl.when(s == NS - 1)
        def _():
            pltpu.sync_copy(z_hbm.at[tail], acc_sh.at[tail])

        plsc.subcore_barrier()

        def do_chunk(k):
            pltpu.sync_copy(sd_hbm.at[k], sd_v)

            @pl.when(c == 0)
            def _():
                pltpu.sync_copy(h0_hbm.at[sd_v.at[0]], rows_v)

            @pl.when(c == 1)
            def _():
                pltpu.sync_copy(h1_hbm.at[sd_v.at[0]], rows_v)

            pltpu.sync_copy(rows_v, acc_sh.at[sd_v.at[1]], add=True)

        @pl.loop(0, SEG_PER)
        def _(j):
            do_chunk(s * SEG_PER + j)

        @pl.when(s < SEG_EXTRA)
        def _():
            do_chunk(NS * SEG_PER + s)

        plsc.subcore_barrier()

        @pl.when(c == 0)
        def _():
            pltpu.sync_copy(acc_sh.at[rows], t0_hbm.at[rows])

            @pl.when(s == NS - 1)
            def _():
                pltpu.sync_copy(acc_sh.at[tail], t0_hbm.at[tail])

        @pl.when(c == 1)
        def _():
            pltpu.sync_copy(acc_sh.at[rows], t1_hbm.at[rows])

            @pl.when(s == NS - 1)
            def _():
                pltpu.sync_copy(acc_sh.at[tail], t1_hbm.at[tail])

    return seg_kernel(hs0, hs1, sd, zeros_nh)


def _tc_step0(x0, wx, bc):
    def body(x_ref, wx_ref, b_ref, h_ref, c_ref):
        z = x_ref[...] * wx_ref[...] + b_ref[...]
        i = jax.nn.sigmoid(z[:, 0:HID])
        t = jnp.tanh(z[:, 2 * HID:3 * HID])
        o = jax.nn.sigmoid(z[:, 3 * HID:4 * HID])
        cc = i * t
        c_ref[...] = cc
        h_ref[...] = o * jnp.tanh(cc)

    return pl.pallas_call(
        body,
        grid=(GRID,),
        in_specs=[
            pl.BlockSpec((RB, 1), lambda i: (i, 0)),
            pl.BlockSpec((1, 4 * HID), lambda i: (0, 0)),
            pl.BlockSpec((1, 4 * HID), lambda i: (0, 0)),
        ],
        out_specs=[pl.BlockSpec((RB, HID), lambda i: (i, 0))] * 2,
        out_shape=[jax.ShapeDtypeStruct((N, HID), jnp.float32)] * 2,
    )(x0, wx, bc)


def _tc_prep(h, dega, degb):
    def body(h_ref, da_ref, db_ref, dinv_ref, h0_ref, h1_ref):
        deg = da_ref[...][:, 0:1] + db_ref[...][:, 0:1]
        dinv = jnp.where(deg > 0, 1.0 / jnp.sqrt(jnp.maximum(deg, 1e-12)), 0.0)
        dinv_ref[...] = dinv
        hs = h_ref[...] * dinv
        h0_ref[...] = hs[:, 0:NH]
        h1_ref[...] = hs[:, NH:HID]

    return pl.pallas_call(
        body,
        grid=(GRID,),
        in_specs=[
            pl.BlockSpec((RB, HID), lambda i: (i, 0)),
            pl.BlockSpec((RB, NH), lambda i: (i, 0)),
            pl.BlockSpec((RB, NH), lambda i: (i, 0)),
        ],
        out_specs=[
            pl.BlockSpec((RB, 1), lambda i: (i, 0)),
            pl.BlockSpec((RB, NH), lambda i: (i, 0)),
            pl.BlockSpec((RB, NH), lambda i: (i, 0)),
        ],
        out_shape=[
            jax.ShapeDtypeStruct((N, 1), jnp.float32),
            jax.ShapeDtypeStruct((N, NH), jnp.float32),
            jax.ShapeDtypeStruct((N, NH), jnp.float32),
        ],
    )(h, dega, degb)


def _tc_step(xt, h, c, tx0, tx1, dinv, w0, w1a, w1b, wx, bc):
    def body(x_ref, h_ref, c_ref, t0_ref, t1_ref, dv_ref,
             w0_ref, w1a_ref, w1b_ref, wx_ref, b_ref,
             hn_ref, cn_ref, h0_ref, h1_ref):
        dv = dv_ref[...]
        nd = -dv
        z = jnp.dot(h_ref[...], w0_ref[...], preferred_element_type=jnp.float32)
        z = z + jnp.dot(t0_ref[...] * nd, w1a_ref[...],
                        preferred_element_type=jnp.float32)
        z = z + jnp.dot(t1_ref[...] * nd, w1b_ref[...],
                        preferred_element_type=jnp.float32)
        z = z + x_ref[...] * wx_ref[...] + b_ref[...]
        i = jax.nn.sigmoid(z[:, 0:HID])
        f = jax.nn.sigmoid(z[:, HID:2 * HID])
        t = jnp.tanh(z[:, 2 * HID:3 * HID])
        o = jax.nn.sigmoid(z[:, 3 * HID:4 * HID])
        cn = f * c_ref[...] + i * t
        hn = o * jnp.tanh(cn)
        hn_ref[...] = hn
        cn_ref[...] = cn
        hs = hn * dv
        h0_ref[...] = hs[:, 0:NH]
        h1_ref[...] = hs[:, NH:HID]

    return pl.pallas_call(
        body,
        grid=(GRID,),
        in_specs=[
            pl.BlockSpec((RB, 1), lambda i: (i, 0)),
            pl.BlockSpec((RB, HID), lambda i: (i, 0)),
            pl.BlockSpec((RB, HID), lambda i: (i, 0)),
            pl.BlockSpec((RB, NH), lambda i: (i, 0)),
            pl.BlockSpec((RB, NH), lambda i: (i, 0)),
            pl.BlockSpec((RB, 1), lambda i: (i, 0)),
            pl.BlockSpec((HID, 4 * HID), lambda i: (0, 0)),
            pl.BlockSpec((NH, 4 * HID), lambda i: (0, 0)),
            pl.BlockSpec((NH, 4 * HID), lambda i: (0, 0)),
            pl.BlockSpec((1, 4 * HID), lambda i: (0, 0)),
            pl.BlockSpec((1, 4 * HID), lambda i: (0, 0)),
        ],
        out_specs=[
            pl.BlockSpec((RB, HID), lambda i: (i, 0)),
            pl.BlockSpec((RB, HID), lambda i: (i, 0)),
            pl.BlockSpec((RB, NH), lambda i: (i, 0)),
            pl.BlockSpec((RB, NH), lambda i: (i, 0)),
        ],
        out_shape=[
            jax.ShapeDtypeStruct((N, HID), jnp.float32),
            jax.ShapeDtypeStruct((N, HID), jnp.float32),
            jax.ShapeDtypeStruct((N, NH), jnp.float32),
            jax.ShapeDtypeStruct((N, NH), jnp.float32),
        ],
    )(xt, h, c, tx0, tx1, dinv, w0, w1a, w1b, wx, bc)


def kernel(x_seq, edge_index, W_i, b_i, Wch_i, bch_i, W_f, b_f, Wch_f, bch_f,
           W_c, b_c, Wch_c, bch_c, W_o, b_o, Wch_o, bch_o):
    src = edge_index[0]
    dst = edge_index[1]

    w0 = jnp.concatenate([Wch_i[0], Wch_f[0], Wch_c[0], Wch_o[0]], axis=1)
    w1 = jnp.concatenate([Wch_i[1], Wch_f[1], Wch_c[1], Wch_o[1]], axis=1)
    w1a = w1[0:NH]
    w1b = w1[NH:HID]
    wx = jnp.concatenate([W_i, W_f, W_c, W_o], axis=1)
    bc = jnp.concatenate([
        b_i + bch_i[None, :], b_f + bch_f[None, :],
        b_c + bch_c[None, :], b_o + bch_o[None, :]], axis=1)

    zeros_nh = jnp.zeros((N, NH), jnp.float32)
    ones_nh = jnp.ones((CH, NH), jnp.float32)

    dega, degb = _sc_degree(src, zeros_nh, ones_nh)
    h, cst = _tc_step0(x_seq[0], wx, bc)
    dinv, hs0, hs1 = _tc_prep(h, dega, degb)

    sd = jnp.stack([src.reshape(NCHUNK, CH), dst.reshape(NCHUNK, CH)], axis=1)

    for t in range(1, 3):
        tx0, tx1 = _sc_segsum(hs0, hs1, sd, zeros_nh)
        h, cst, hs0, hs1 = _tc_step(
            x_seq[t], h, cst, tx0, tx1, dinv, w0, w1a, w1b, wx, bc)

    return h


# speedup vs baseline: 1.4390x; 1.0001x over previous
import functools

import jax
import jax.numpy as jnp
from jax import lax
from jax.experimental import pallas as pl
from jax.experimental.pallas import tpu as pltpu
from jax.experimental.pallas import tpu_sc as plsc

N = 10000
E = 160000
HID = 256
NH = HID // 2
CH = 128
NCHUNK = E // CH
NC = 2
NS = 16
ROWS_PER_SUB = 624
ROWS_TAIL = N - NS * ROWS_PER_SUB

DEG_PER = NCHUNK // (NC * NS)
DEG_EXTRA = NCHUNK - DEG_PER * NC * NS
SEG_PER = NCHUNK // NS
SEG_EXTRA = NCHUNK - SEG_PER * NS

RB = 2000
GRID = N // RB

def _sc_mesh():
    return plsc.VectorSubcoreMesh(
        core_axis_name="c", subcore_axis_name="s", num_cores=NC, num_subcores=NS)


def _sc_degree(src, zeros_nh, ones_nh):
    @functools.partial(
        pl.kernel,
        out_type=[jax.ShapeDtypeStruct((N, NH), jnp.float32)] * 2,
        mesh=_sc_mesh(),
        scratch_types=[
            pltpu.VMEM((CH,), jnp.int32),
            pltpu.VMEM((CH, NH), jnp.float32),
            pltpu.VMEM_SHARED((N, NH), jnp.float32),
        ],
    )
    def deg_kernel(src_hbm, z_hbm, ones_hbm, outa_hbm, outb_hbm,
                   idx_v, ones_v, acc_sh):
        c = lax.axis_index("c")
        s = lax.axis_index("s")
        w = c * NS + s
        rows = pl.ds(s * ROWS_PER_SUB, ROWS_PER_SUB)
        tail = pl.ds(NS * ROWS_PER_SUB, ROWS_TAIL)
        pltpu.sync_copy(z_hbm.at[rows], acc_sh.at[rows])

        @pl.when(s == NS - 1)
        def _():
            pltpu.sync_copy(z_hbm.at[tail], acc_sh.at[tail])

        pltpu.sync_copy(ones_hbm, ones_v)
        plsc.subcore_barrier()

        def do_chunk(k):
            off = pl.multiple_of(k * CH, CH)
            pltpu.sync_copy(src_hbm.at[pl.ds(off, CH)], idx_v)
            pltpu.sync_copy(ones_v, acc_sh.at[idx_v], add=True)

        @pl.loop(0, DEG_PER)
        def _(j):
            do_chunk(w * DEG_PER + j)

        @pl.when(w < DEG_EXTRA)
        def _():
            do_chunk(NC * NS * DEG_PER + w)

        plsc.subcore_barrier()

        @pl.when(c == 0)
        def _():
            pltpu.sync_copy(acc_sh.at[rows], outa_hbm.at[rows])

            @pl.when(s == NS - 1)
            def _():
                pltpu.sync_copy(acc_sh.at[tail], outa_hbm.at[tail])

        @pl.when(c == 1)
        def _():
            pltpu.sync_copy(acc_sh.at[rows], outb_hbm.at[rows])

            @pl.when(s == NS - 1)
            def _():
                pltpu.sync_copy(acc_sh.at[tail], outb_hbm.at[tail])

    return deg_kernel(src, zeros_nh, ones_nh)


def _sc_segsum(hs0, hs1, sd, zeros_nh):
    @functools.partial(
        pl.kernel,
        out_type=[jax.ShapeDtypeStruct((N, NH), jnp.float32)] * 2,
        mesh=_sc_mesh(),
        scratch_types=[
            pltpu.VMEM((2, CH), jnp.int32),
            pltpu.VMEM((CH, NH), jnp.float32),
            pltpu.VMEM_SHARED((N, NH), jnp.float32),
        ],
    )
    def seg_kernel(h0_hbm, h1_hbm, sd_hbm, z_hbm, t0_hbm, t1_hbm,
                   sd_v, rows_v, acc_sh):
        c = lax.axis_index("c")
        s = lax.axis_index("s")
        rows = pl.ds(s * ROWS_PER_SUB, ROWS_PER_SUB)
        tail = pl.ds(NS * ROWS_PER_SUB, ROWS_TAIL)
        pltpu.sync_copy(z_hbm.at[rows], acc_sh.at[rows])

        @pl.when(s == NS - 1)
        def _():
            pltpu.sync_copy(z_hbm.at[tail], acc_sh.at[tail])

        plsc.subcore_barrier()

        def do_chunk(k):
            pltpu.sync_copy(sd_hbm.at[k], sd_v)

            @pl.when(c == 0)
            def _():
                pltpu.sync_copy(h0_hbm.at[sd_v.at[0]], rows_v)

            @pl.when(c == 1)
            def _():
                pltpu.sync_copy(h1_hbm.at[sd_v.at[0]], rows_v)

            pltpu.sync_copy(rows_v, acc_sh.at[sd_v.at[1]], add=True)

        @pl.loop(0, SEG_PER)
        def _(j):
            do_chunk(s * SEG_PER + j)

        @pl.when(s < SEG_EXTRA)
        def _():
            do_chunk(NS * SEG_PER + s)

        plsc.subcore_barrier()

        @pl.when(c == 0)
        def _():
            pltpu.sync_copy(acc_sh.at[rows], t0_hbm.at[rows])

            @pl.when(s == NS - 1)
            def _():
                pltpu.sync_copy(acc_sh.at[tail], t0_hbm.at[tail])

        @pl.when(c == 1)
        def _():
            pltpu.sync_copy(acc_sh.at[rows], t1_hbm.at[rows])

            @pl.when(s == NS - 1)
            def _():
                pltpu.sync_copy(acc_sh.at[tail], t1_hbm.at[tail])

    return seg_kernel(hs0, hs1, sd, zeros_nh)


def _tc_step0(x0, wx, bc):
    def body(x_ref, wx_ref, b_ref, h_ref, c_ref):
        z = x_ref[...] * wx_ref[...] + b_ref[...]
        i = jax.nn.sigmoid(z[:, 0:HID])
        t = jnp.tanh(z[:, 2 * HID:3 * HID])
        o = jax.nn.sigmoid(z[:, 3 * HID:4 * HID])
        cc = i * t
        c_ref[...] = cc
        h_ref[...] = o * jnp.tanh(cc)

    return pl.pallas_call(
        body,
        grid=(GRID,),
        in_specs=[
            pl.BlockSpec((RB, 1), lambda i: (i, 0)),
            pl.BlockSpec((1, 4 * HID), lambda i: (0, 0)),
            pl.BlockSpec((1, 4 * HID), lambda i: (0, 0)),
        ],
        out_specs=[pl.BlockSpec((RB, HID), lambda i: (i, 0))] * 2,
        out_shape=[jax.ShapeDtypeStruct((N, HID), jnp.float32)] * 2,
    )(x0, wx, bc)


def _tc_prep(h, dega, degb):
    def body(h_ref, da_ref, db_ref, dinv_ref, h0_ref, h1_ref):
        deg = da_ref[...][:, 0:1] + db_ref[...][:, 0:1]
        dinv = jnp.where(deg > 0, 1.0 / jnp.sqrt(jnp.maximum(deg, 1e-12)), 0.0)
        dinv_ref[...] = dinv
        hs = h_ref[...] * dinv
        h0_ref[...] = hs[:, 0:NH]
        h1_ref[...] = hs[:, NH:HID]

    return pl.pallas_call(
        body,
        grid=(GRID,),
        in_specs=[
            pl.BlockSpec((RB, HID), lambda i: (i, 0)),
            pl.BlockSpec((RB, NH), lambda i: (i, 0)),
            pl.BlockSpec((RB, NH), lambda i: (i, 0)),
        ],
        out_specs=[
            pl.BlockSpec((RB, 1), lambda i: (i, 0)),
            pl.BlockSpec((RB, NH), lambda i: (i, 0)),
            pl.BlockSpec((RB, NH), lambda i: (i, 0)),
        ],
        out_shape=[
            jax.ShapeDtypeStruct((N, 1), jnp.float32),
            jax.ShapeDtypeStruct((N, NH), jnp.float32),
            jax.ShapeDtypeStruct((N, NH), jnp.float32),
        ],
    )(h, dega, degb)


def _tc_step(xt, h, c, tx0, tx1, dinv, w0, w1a, w1b, wx, bc):
    def body(x_ref, h_ref, c_ref, t0_ref, t1_ref, dv_ref,
             w0_ref, w1a_ref, w1b_ref, wx_ref, b_ref,
             hn_ref, cn_ref, h0_ref, h1_ref):
        dv = dv_ref[...]
        nd = -dv
        hb = h_ref[...].astype(jnp.bfloat16)
        t0b = (t0_ref[...] * nd).astype(jnp.bfloat16)
        t1b = (t1_ref[...] * nd).astype(jnp.bfloat16)
        z = jnp.dot(hb, w0_ref[...], preferred_element_type=jnp.float32)
        z = z + jnp.dot(t0b, w1a_ref[...], preferred_element_type=jnp.float32)
        z = z + jnp.dot(t1b, w1b_ref[...], preferred_element_type=jnp.float32)
        z = z + x_ref[...] * wx_ref[...] + b_ref[...]
        i = jax.nn.sigmoid(z[:, 0:HID])
        f = jax.nn.sigmoid(z[:, HID:2 * HID])
        t = jnp.tanh(z[:, 2 * HID:3 * HID])
        o = jax.nn.sigmoid(z[:, 3 * HID:4 * HID])
        cn = f * c_ref[...] + i * t
        hn = o * jnp.tanh(cn)
        hn_ref[...] = hn
        cn_ref[...] = cn
        hs = hn * dv
        h0_ref[...] = hs[:, 0:NH]
        h1_ref[...] = hs[:, NH:HID]

    return pl.pallas_call(
        body,
        grid=(GRID,),
        in_specs=[
            pl.BlockSpec((RB, 1), lambda i: (i, 0)),
            pl.BlockSpec((RB, HID), lambda i: (i, 0)),
            pl.BlockSpec((RB, HID), lambda i: (i, 0)),
            pl.BlockSpec((RB, NH), lambda i: (i, 0)),
            pl.BlockSpec((RB, NH), lambda i: (i, 0)),
            pl.BlockSpec((RB, 1), lambda i: (i, 0)),
            pl.BlockSpec((HID, 4 * HID), lambda i: (0, 0)),
            pl.BlockSpec((NH, 4 * HID), lambda i: (0, 0)),
            pl.BlockSpec((NH, 4 * HID), lambda i: (0, 0)),
            pl.BlockSpec((1, 4 * HID), lambda i: (0, 0)),
            pl.BlockSpec((1, 4 * HID), lambda i: (0, 0)),
        ],
        out_specs=[
            pl.BlockSpec((RB, HID), lambda i: (i, 0)),
            pl.BlockSpec((RB, HID), lambda i: (i, 0)),
            pl.BlockSpec((RB, NH), lambda i: (i, 0)),
            pl.BlockSpec((RB, NH), lambda i: (i, 0)),
        ],
        out_shape=[
            jax.ShapeDtypeStruct((N, HID), jnp.float32),
            jax.ShapeDtypeStruct((N, HID), jnp.float32),
            jax.ShapeDtypeStruct((N, NH), jnp.float32),
            jax.ShapeDtypeStruct((N, NH), jnp.float32),
        ],
    )(xt, h, c, tx0, tx1, dinv, w0, w1a, w1b, wx, bc)


def kernel(x_seq, edge_index, W_i, b_i, Wch_i, bch_i, W_f, b_f, Wch_f, bch_f,
           W_c, b_c, Wch_c, bch_c, W_o, b_o, Wch_o, bch_o):
    src = edge_index[0]
    dst = edge_index[1]

    w0 = jnp.concatenate(
        [Wch_i[0], Wch_f[0], Wch_c[0], Wch_o[0]], axis=1).astype(jnp.bfloat16)
    w1 = jnp.concatenate([Wch_i[1], Wch_f[1], Wch_c[1], Wch_o[1]], axis=1)
    w1a = w1[0:NH].astype(jnp.bfloat16)
    w1b = w1[NH:HID].astype(jnp.bfloat16)
    wx = jnp.concatenate([W_i, W_f, W_c, W_o], axis=1)
    bc = jnp.concatenate([
        b_i + bch_i[None, :], b_f + bch_f[None, :],
        b_c + bch_c[None, :], b_o + bch_o[None, :]], axis=1)

    zeros_nh = jnp.zeros((N, NH), jnp.float32)
    ones_nh = jnp.ones((CH, NH), jnp.float32)

    dega, degb = _sc_degree(src, zeros_nh, ones_nh)
    h, cst = _tc_step0(x_seq[0], wx, bc)
    dinv, hs0, hs1 = _tc_prep(h, dega, degb)

    sd = jnp.stack([src.reshape(NCHUNK, CH), dst.reshape(NCHUNK, CH)], axis=1)

    for t in range(1, 3):
        tx0, tx1 = _sc_segsum(hs0, hs1, sd, zeros_nh)
        h, cst, hs0, hs1 = _tc_step(
            x_seq[t], h, cst, tx0, tx1, dinv, w0, w1a, w1b, wx, bc)

    return h


# speedup vs baseline: 1.4800x; 1.0285x over previous
import functools

import jax
import jax.numpy as jnp
from jax import lax
from jax.experimental import pallas as pl
from jax.experimental.pallas import tpu as pltpu
from jax.experimental.pallas import tpu_sc as plsc

N = 10000
E = 160000
HID = 256
NH = HID // 2
CH = 128
NCHUNK = E // CH
NC = 2
NS = 16
ROWS_PER_SUB = 624
ROWS_TAIL = N - NS * ROWS_PER_SUB

DEGC = 40
DEG_EPAD = NC * NS * DEGC * CH
SEG_PER = NCHUNK // NS
SEG_EXTRA = NCHUNK - SEG_PER * NS

RB = 2000
GRID = N // RB

def _sc_mesh():
    return plsc.VectorSubcoreMesh(
        core_axis_name="c", subcore_axis_name="s", num_cores=NC, num_subcores=NS)


def _sc_degree(srcd, zeros_nh, ones_nh):
    @functools.partial(
        pl.kernel,
        out_type=[jax.ShapeDtypeStruct((N, NH), jnp.float32)] * 2,
        mesh=_sc_mesh(),
        scratch_types=[
            pltpu.VMEM((DEGC, CH), jnp.int32),
            pltpu.VMEM((CH, NH), jnp.float32),
            pltpu.VMEM_SHARED((N, NH), jnp.float32),
            pltpu.SemaphoreType.DMA,
        ],
    )
    def deg_kernel(src_hbm, z_hbm, ones_hbm, outa_hbm, outb_hbm,
                   si_d, ones_v, acc_sh, sem):
        c = lax.axis_index("c")
        s = lax.axis_index("s")
        w = c * NS + s
        rows = pl.ds(s * ROWS_PER_SUB, ROWS_PER_SUB)
        tail = pl.ds(NS * ROWS_PER_SUB, ROWS_TAIL)
        pltpu.sync_copy(z_hbm.at[rows], acc_sh.at[rows])

        @pl.when(s == NS - 1)
        def _():
            pltpu.sync_copy(z_hbm.at[tail], acc_sh.at[tail])

        pltpu.sync_copy(src_hbm.at[pl.ds(w * DEGC, DEGC)], si_d)
        pltpu.sync_copy(ones_hbm, ones_v)
        plsc.subcore_barrier()

        @pl.loop(0, DEGC)
        def _(k):
            pltpu.async_copy(ones_v, acc_sh.at[si_d.at[k]], sem, add=True)

        @pl.loop(0, DEGC)
        def _(k):
            pltpu.make_async_copy(ones_v, acc_sh.at[si_d.at[0]], sem).wait()

        plsc.subcore_barrier()

        @pl.when(c == 0)
        def _():
            pltpu.sync_copy(acc_sh.at[rows], outa_hbm.at[rows])

            @pl.when(s == NS - 1)
            def _():
                pltpu.sync_copy(acc_sh.at[tail], outa_hbm.at[tail])

        @pl.when(c == 1)
        def _():
            pltpu.sync_copy(acc_sh.at[rows], outb_hbm.at[rows])

            @pl.when(s == NS - 1)
            def _():
                pltpu.sync_copy(acc_sh.at[tail], outb_hbm.at[tail])

    return deg_kernel(srcd, zeros_nh, ones_nh)


def _sc_segsum(hs0, hs1, sd, zeros_nh):
    @functools.partial(
        pl.kernel,
        out_type=[jax.ShapeDtypeStruct((N, NH), jnp.float32)] * 2,
        mesh=_sc_mesh(),
        scratch_types=[
            pltpu.VMEM((2, CH), jnp.int32),
            pltpu.VMEM((CH, NH), jnp.float32),
            pltpu.VMEM_SHARED((N, NH), jnp.float32),
        ],
    )
    def seg_kernel(h0_hbm, h1_hbm, sd_hbm, z_hbm, t0_hbm, t1_hbm,
                   sd_v, rows_v, acc_sh):
        c = lax.axis_index("c")
        s = lax.axis_index("s")
        rows = pl.ds(s * ROWS_PER_SUB, ROWS_PER_SUB)
        tail = pl.ds(NS * ROWS_PER_SUB, ROWS_TAIL)
        pltpu.sync_copy(z_hbm.at[rows], acc_sh.at[rows])

        @pl.when(s == NS - 1)
        def _():
            pltpu.sync_copy(z_hbm.at[tail], acc_sh.at[tail])

        plsc.subcore_barrier()

        def do_chunk(k):
            pltpu.sync_copy(sd_hbm.at[k], sd_v)

            @pl.when(c == 0)
            def _():
                pltpu.sync_copy(h0_hbm.at[sd_v.at[0]], rows_v)

            @pl.when(c == 1)
            def _():
                pltpu.sync_copy(h1_hbm.at[sd_v.at[0]], rows_v)

            pltpu.sync_copy(rows_v, acc_sh.at[sd_v.at[1]], add=True)

        @pl.loop(0, SEG_PER)
        def _(j):
            do_chunk(s * SEG_PER + j)

        @pl.when(s < SEG_EXTRA)
        def _():
            do_chunk(NS * SEG_PER + s)

        plsc.subcore_barrier()

        @pl.when(c == 0)
        def _():
            pltpu.sync_copy(acc_sh.at[rows], t0_hbm.at[rows])

            @pl.when(s == NS - 1)
            def _():
                pltpu.sync_copy(acc_sh.at[tail], t0_hbm.at[tail])

        @pl.when(c == 1)
        def _():
            pltpu.sync_copy(acc_sh.at[rows], t1_hbm.at[rows])

            @pl.when(s == NS - 1)
            def _():
                pltpu.sync_copy(acc_sh.at[tail], t1_hbm.at[tail])

    return seg_kernel(hs0, hs1, sd, zeros_nh)


def _tc_step0(x0, wx, bc):
    def body(x_ref, wx_ref, b_ref, h_ref, c_ref):
        z = x_ref[...] * wx_ref[...] + b_ref[...]
        i = jax.nn.sigmoid(z[:, 0:HID])
        t = jnp.tanh(z[:, 2 * HID:3 * HID])
        o = jax.nn.sigmoid(z[:, 3 * HID:4 * HID])
        cc = i * t
        c_ref[...] = cc
        h_ref[...] = o * jnp.tanh(cc)

    return pl.pallas_call(
        body,
        grid=(GRID,),
        in_specs=[
            pl.BlockSpec((RB, 1), lambda i: (i, 0)),
            pl.BlockSpec((1, 4 * HID), lambda i: (0, 0)),
            pl.BlockSpec((1, 4 * HID), lambda i: (0, 0)),
        ],
        out_specs=[pl.BlockSpec((RB, HID), lambda i: (i, 0))] * 2,
        out_shape=[jax.ShapeDtypeStruct((N, HID), jnp.float32)] * 2,
    )(x0, wx, bc)


def _tc_prep(h, dega, degb, padc):
    def body(h_ref, da_ref, db_ref, pc_ref, dinv_ref, h0_ref, h1_ref):
        deg = da_ref[...][:, 0:1] + db_ref[...][:, 0:1] - pc_ref[...]
        dinv = jnp.where(deg > 0, 1.0 / jnp.sqrt(jnp.maximum(deg, 1e-12)), 0.0)
        dinv_ref[...] = dinv
        hs = h_ref[...] * dinv
        h0_ref[...] = hs[:, 0:NH]
        h1_ref[...] = hs[:, NH:HID]

    return pl.pallas_call(
        body,
        grid=(GRID,),
        in_specs=[
            pl.BlockSpec((RB, HID), lambda i: (i, 0)),
            pl.BlockSpec((RB, NH), lambda i: (i, 0)),
            pl.BlockSpec((RB, NH), lambda i: (i, 0)),
            pl.BlockSpec((RB, 1), lambda i: (i, 0)),
        ],
        out_specs=[
            pl.BlockSpec((RB, 1), lambda i: (i, 0)),
            pl.BlockSpec((RB, NH), lambda i: (i, 0)),
            pl.BlockSpec((RB, NH), lambda i: (i, 0)),
        ],
        out_shape=[
            jax.ShapeDtypeStruct((N, 1), jnp.float32),
            jax.ShapeDtypeStruct((N, NH), jnp.float32),
            jax.ShapeDtypeStruct((N, NH), jnp.float32),
        ],
    )(h, dega, degb, padc)


def _tc_step(xt, h, c, tx0, tx1, dinv, w0, w1a, w1b, wx, bc):
    def body(x_ref, h_ref, c_ref, t0_ref, t1_ref, dv_ref,
             w0_ref, w1a_ref, w1b_ref, wx_ref, b_ref,
             hn_ref, cn_ref, h0_ref, h1_ref):
        dv = dv_ref[...]
        nd = -dv
        z = jnp.dot(h_ref[...], w0_ref[...], preferred_element_type=jnp.float32)
        z = z + jnp.dot(t0_ref[...] * nd, w1a_ref[...],
                        preferred_element_type=jnp.float32)
        z = z + jnp.dot(t1_ref[...] * nd, w1b_ref[...],
                        preferred_element_type=jnp.float32)
        z = z + x_ref[...] * wx_ref[...] + b_ref[...]
        i = jax.nn.sigmoid(z[:, 0:HID])
        f = jax.nn.sigmoid(z[:, HID:2 * HID])
        t = jnp.tanh(z[:, 2 * HID:3 * HID])
        o = jax.nn.sigmoid(z[:, 3 * HID:4 * HID])
        cn = f * c_ref[...] + i * t
        hn = o * jnp.tanh(cn)
        hn_ref[...] = hn
        cn_ref[...] = cn
        hs = hn * dv
        h0_ref[...] = hs[:, 0:NH]
        h1_ref[...] = hs[:, NH:HID]

    return pl.pallas_call(
        body,
        grid=(GRID,),
        in_specs=[
            pl.BlockSpec((RB, 1), lambda i: (i, 0)),
            pl.BlockSpec((RB, HID), lambda i: (i, 0)),
            pl.BlockSpec((RB, HID), lambda i: (i, 0)),
            pl.BlockSpec((RB, NH), lambda i: (i, 0)),
            pl.BlockSpec((RB, NH), lambda i: (i, 0)),
            pl.BlockSpec((RB, 1), lambda i: (i, 0)),
            pl.BlockSpec((HID, 4 * HID), lambda i: (0, 0)),
            pl.BlockSpec((NH, 4 * HID), lambda i: (0, 0)),
            pl.BlockSpec((NH, 4 * HID), lambda i: (0, 0)),
            pl.BlockSpec((1, 4 * HID), lambda i: (0, 0)),
            pl.BlockSpec((1, 4 * HID), lambda i: (0, 0)),
        ],
        out_specs=[
            pl.BlockSpec((RB, HID), lambda i: (i, 0)),
            pl.BlockSpec((RB, HID), lambda i: (i, 0)),
            pl.BlockSpec((RB, NH), lambda i: (i, 0)),
            pl.BlockSpec((RB, NH), lambda i: (i, 0)),
        ],
        out_shape=[
            jax.ShapeDtypeStruct((N, HID), jnp.float32),
            jax.ShapeDtypeStruct((N, HID), jnp.float32),
            jax.ShapeDtypeStruct((N, NH), jnp.float32),
            jax.ShapeDtypeStruct((N, NH), jnp.float32),
        ],
    )(xt, h, c, tx0, tx1, dinv, w0, w1a, w1b, wx, bc)


def kernel(x_seq, edge_index, W_i, b_i, Wch_i, bch_i, W_f, b_f, Wch_f, bch_f,
           W_c, b_c, Wch_c, bch_c, W_o, b_o, Wch_o, bch_o):
    src = edge_index[0]
    dst = edge_index[1]

    w0 = jnp.concatenate([Wch_i[0], Wch_f[0], Wch_c[0], Wch_o[0]], axis=1)
    w1 = jnp.concatenate([Wch_i[1], Wch_f[1], Wch_c[1], Wch_o[1]], axis=1)
    w1a = w1[0:NH]
    w1b = w1[NH:HID]
    wx = jnp.concatenate([W_i, W_f, W_c, W_o], axis=1)
    bc = jnp.concatenate([
        b_i + bch_i[None, :], b_f + bch_f[None, :],
        b_c + bch_c[None, :], b_o + bch_o[None, :]], axis=1)

    zeros_nh = jnp.zeros((N, NH), jnp.float32)
    ones_nh = jnp.ones((CH, NH), jnp.float32)

    srcd = jnp.concatenate(
        [src, jnp.zeros((DEG_EPAD - E,), jnp.int32)]).reshape(NC * NS * DEGC, CH)
    padc = jnp.concatenate([
        jnp.full((1, 1), float(DEG_EPAD - E), jnp.float32),
        jnp.zeros((N - 1, 1), jnp.float32)])

    dega, degb = _sc_degree(srcd, zeros_nh, ones_nh)
    h, cst = _tc_step0(x_seq[0], wx, bc)
    dinv, hs0, hs1 = _tc_prep(h, dega, degb, padc)

    sd = jnp.stack([src.reshape(NCHUNK, CH), dst.reshape(NCHUNK, CH)], axis=1)

    for t in range(1, 3):
        tx0, tx1 = _sc_segsum(hs0, hs1, sd, zeros_nh)
        h, cst, hs0, hs1 = _tc_step(
            x_seq[t], h, cst, tx0, tx1, dinv, w0, w1a, w1b, wx, bc)

    return h


# speedup vs baseline: 1.7686x; 1.1950x over previous
import functools

import jax
import jax.numpy as jnp
from jax import lax
from jax.experimental import pallas as pl
from jax.experimental.pallas import tpu as pltpu
from jax.experimental.pallas import tpu_sc as plsc

N = 10000
E = 160000
HID = 256
NH = HID // 2
CH = 128
NCHUNK = E // CH
NC = 2
NS = 16
ROWS_PER_SUB = 624
ROWS_TAIL = N - NS * ROWS_PER_SUB

DEGC = 40
DEG_EPAD = NC * NS * DEGC * CH
SEG_PER = NCHUNK // NS
SEG_EXTRA = NCHUNK - SEG_PER * NS

RB = 2000
GRID = N // RB

def _sc_mesh():
    return plsc.VectorSubcoreMesh(
        core_axis_name="c", subcore_axis_name="s", num_cores=NC, num_subcores=NS)


def _sc_degree(srcd, zeros_nh, ones_nh):
    @functools.partial(
        pl.kernel,
        out_type=[jax.ShapeDtypeStruct((N, NH), jnp.float32)] * 2,
        mesh=_sc_mesh(),
        scratch_types=[
            pltpu.VMEM((DEGC, CH), jnp.int32),
            pltpu.VMEM((CH, NH), jnp.float32),
            pltpu.VMEM_SHARED((N, NH), jnp.float32),
            pltpu.SemaphoreType.DMA,
        ],
    )
    def deg_kernel(src_hbm, z_hbm, ones_hbm, outa_hbm, outb_hbm,
                   si_d, ones_v, acc_sh, sem):
        c = lax.axis_index("c")
        s = lax.axis_index("s")
        w = c * NS + s
        rows = pl.ds(s * ROWS_PER_SUB, ROWS_PER_SUB)
        tail = pl.ds(NS * ROWS_PER_SUB, ROWS_TAIL)
        pltpu.sync_copy(z_hbm.at[rows], acc_sh.at[rows])

        @pl.when(s == NS - 1)
        def _():
            pltpu.sync_copy(z_hbm.at[tail], acc_sh.at[tail])

        pltpu.sync_copy(src_hbm.at[pl.ds(w * DEGC, DEGC)], si_d)
        pltpu.sync_copy(ones_hbm, ones_v)
        plsc.subcore_barrier()

        @pl.loop(0, DEGC)
        def _(k):
            pltpu.async_copy(ones_v, acc_sh.at[si_d.at[k]], sem, add=True)

        @pl.loop(0, DEGC)
        def _(k):
            pltpu.make_async_copy(ones_v, acc_sh.at[si_d.at[0]], sem).wait()

        plsc.subcore_barrier()

        @pl.when(c == 0)
        def _():
            pltpu.sync_copy(acc_sh.at[rows], outa_hbm.at[rows])

            @pl.when(s == NS - 1)
            def _():
                pltpu.sync_copy(acc_sh.at[tail], outa_hbm.at[tail])

        @pl.when(c == 1)
        def _():
            pltpu.sync_copy(acc_sh.at[rows], outb_hbm.at[rows])

            @pl.when(s == NS - 1)
            def _():
                pltpu.sync_copy(acc_sh.at[tail], outb_hbm.at[tail])

    return deg_kernel(srcd, zeros_nh, ones_nh)


def _sc_segsum(hs0, hs1, sd, zeros_nh):
    @functools.partial(
        pl.kernel,
        out_type=[jax.ShapeDtypeStruct((N, NH), jnp.float32)] * 2,
        mesh=_sc_mesh(),
        scratch_types=[
            pltpu.VMEM((2, CH), jnp.int32),
            pltpu.VMEM((2, CH), jnp.int32),
            pltpu.VMEM((CH, NH), jnp.float32),
            pltpu.VMEM((CH, NH), jnp.float32),
            pltpu.VMEM_SHARED((N, NH), jnp.float32),
            pltpu.SemaphoreType.DMA,
            pltpu.SemaphoreType.DMA,
        ],
    )
    def seg_kernel(h0_hbm, h1_hbm, sd_hbm, z_hbm, t0_hbm, t1_hbm,
                   sd0_v, sd1_v, rows0_v, rows1_v, acc_sh, sem0, sem1):
        c = lax.axis_index("c")
        s = lax.axis_index("s")
        rows = pl.ds(s * ROWS_PER_SUB, ROWS_PER_SUB)
        tail = pl.ds(NS * ROWS_PER_SUB, ROWS_TAIL)
        pltpu.sync_copy(z_hbm.at[rows], acc_sh.at[rows])

        @pl.when(s == NS - 1)
        def _():
            pltpu.sync_copy(z_hbm.at[tail], acc_sh.at[tail])

        plsc.subcore_barrier()

        sds = (sd0_v, sd1_v)
        rows_b = (rows0_v, rows1_v)
        sems = (sem0, sem1)

        def wait_s(b):
            pltpu.make_async_copy(
                rows_b[b], acc_sh.at[sds[b].at[1]], sems[b]).wait()

        def do_chunk(k, b):
            pltpu.sync_copy(sd_hbm.at[k], sds[b])

            @pl.when(c == 0)
            def _():
                pltpu.sync_copy(h0_hbm.at[sds[b].at[0]], rows_b[b])

            @pl.when(c == 1)
            def _():
                pltpu.sync_copy(h1_hbm.at[sds[b].at[0]], rows_b[b])

            pltpu.async_copy(rows_b[b], acc_sh.at[sds[b].at[1]], sems[b],
                             add=True)

        @pl.loop(0, SEG_PER, step=2)
        def _(j):
            for b in range(2):
                @pl.when(j > 0)
                def _():
                    wait_s(b)

                do_chunk(s * SEG_PER + j + b, b)

        @pl.when(s < SEG_EXTRA)
        def _():
            wait_s(0)
            do_chunk(NS * SEG_PER + s, 0)

        wait_s(0)
        wait_s(1)
        plsc.subcore_barrier()

        @pl.when(c == 0)
        def _():
            pltpu.sync_copy(acc_sh.at[rows], t0_hbm.at[rows])

            @pl.when(s == NS - 1)
            def _():
                pltpu.sync_copy(acc_sh.at[tail], t0_hbm.at[tail])

        @pl.when(c == 1)
        def _():
            pltpu.sync_copy(acc_sh.at[rows], t1_hbm.at[rows])

            @pl.when(s == NS - 1)
            def _():
                pltpu.sync_copy(acc_sh.at[tail], t1_hbm.at[tail])

    return seg_kernel(hs0, hs1, sd, zeros_nh)


def _tc_step0(x0, wx, bc):
    def body(x_ref, wx_ref, b_ref, h_ref, c_ref):
        z = x_ref[...] * wx_ref[...] + b_ref[...]
        i = jax.nn.sigmoid(z[:, 0:HID])
        t = jnp.tanh(z[:, 2 * HID:3 * HID])
        o = jax.nn.sigmoid(z[:, 3 * HID:4 * HID])
        cc = i * t
        c_ref[...] = cc
        h_ref[...] = o * jnp.tanh(cc)

    return pl.pallas_call(
        body,
        grid=(GRID,),
        in_specs=[
            pl.BlockSpec((RB, 1), lambda i: (i, 0)),
            pl.BlockSpec((1, 4 * HID), lambda i: (0, 0)),
            pl.BlockSpec((1, 4 * HID), lambda i: (0, 0)),
        ],
        out_specs=[pl.BlockSpec((RB, HID), lambda i: (i, 0))] * 2,
        out_shape=[jax.ShapeDtypeStruct((N, HID), jnp.float32)] * 2,
    )(x0, wx, bc)


def _tc_prep(h, dega, degb, padc):
    def body(h_ref, da_ref, db_ref, pc_ref, dinv_ref, h0_ref, h1_ref):
        deg = da_ref[...][:, 0:1] + db_ref[...][:, 0:1] - pc_ref[...]
        dinv = jnp.where(deg > 0, 1.0 / jnp.sqrt(jnp.maximum(deg, 1e-12)), 0.0)
        dinv_ref[...] = dinv
        hs = h_ref[...] * dinv
        h0_ref[...] = hs[:, 0:NH]
        h1_ref[...] = hs[:, NH:HID]

    return pl.pallas_call(
        body,
        grid=(GRID,),
        in_specs=[
            pl.BlockSpec((RB, HID), lambda i: (i, 0)),
            pl.BlockSpec((RB, NH), lambda i: (i, 0)),
            pl.BlockSpec((RB, NH), lambda i: (i, 0)),
            pl.BlockSpec((RB, 1), lambda i: (i, 0)),
        ],
        out_specs=[
            pl.BlockSpec((RB, 1), lambda i: (i, 0)),
            pl.BlockSpec((RB, NH), lambda i: (i, 0)),
            pl.BlockSpec((RB, NH), lambda i: (i, 0)),
        ],
        out_shape=[
            jax.ShapeDtypeStruct((N, 1), jnp.float32),
            jax.ShapeDtypeStruct((N, NH), jnp.float32),
            jax.ShapeDtypeStruct((N, NH), jnp.float32),
        ],
    )(h, dega, degb, padc)


def _tc_step(xt, h, c, tx0, tx1, dinv, w0, w1a, w1b, wx, bc):
    def body(x_ref, h_ref, c_ref, t0_ref, t1_ref, dv_ref,
             w0_ref, w1a_ref, w1b_ref, wx_ref, b_ref,
             hn_ref, cn_ref, h0_ref, h1_ref):
        dv = dv_ref[...]
        nd = -dv
        z = jnp.dot(h_ref[...], w0_ref[...], preferred_element_type=jnp.float32)
        z = z + jnp.dot(t0_ref[...] * nd, w1a_ref[...],
                        preferred_element_type=jnp.float32)
        z = z + jnp.dot(t1_ref[...] * nd, w1b_ref[...],
                        preferred_element_type=jnp.float32)
        z = z + x_ref[...] * wx_ref[...] + b_ref[...]
        i = jax.nn.sigmoid(z[:, 0:HID])
        f = jax.nn.sigmoid(z[:, HID:2 * HID])
        t = jnp.tanh(z[:, 2 * HID:3 * HID])
        o = jax.nn.sigmoid(z[:, 3 * HID:4 * HID])
        cn = f * c_ref[...] + i * t
        hn = o * jnp.tanh(cn)
        hn_ref[...] = hn
        cn_ref[...] = cn
        hs = hn * dv
        h0_ref[...] = hs[:, 0:NH]
        h1_ref[...] = hs[:, NH:HID]

    return pl.pallas_call(
        body,
        grid=(GRID,),
        in_specs=[
            pl.BlockSpec((RB, 1), lambda i: (i, 0)),
            pl.BlockSpec((RB, HID), lambda i: (i, 0)),
            pl.BlockSpec((RB, HID), lambda i: (i, 0)),
            pl.BlockSpec((RB, NH), lambda i: (i, 0)),
            pl.BlockSpec((RB, NH), lambda i: (i, 0)),
            pl.BlockSpec((RB, 1), lambda i: (i, 0)),
            pl.BlockSpec((HID, 4 * HID), lambda i: (0, 0)),
            pl.BlockSpec((NH, 4 * HID), lambda i: (0, 0)),
            pl.BlockSpec((NH, 4 * HID), lambda i: (0, 0)),
            pl.BlockSpec((1, 4 * HID), lambda i: (0, 0)),
            pl.BlockSpec((1, 4 * HID), lambda i: (0, 0)),
        ],
        out_specs=[
            pl.BlockSpec((RB, HID), lambda i: (i, 0)),
            pl.BlockSpec((RB, HID), lambda i: (i, 0)),
            pl.BlockSpec((RB, NH), lambda i: (i, 0)),
            pl.BlockSpec((RB, NH), lambda i: (i, 0)),
        ],
        out_shape=[
            jax.ShapeDtypeStruct((N, HID), jnp.float32),
            jax.ShapeDtypeStruct((N, HID), jnp.float32),
            jax.ShapeDtypeStruct((N, NH), jnp.float32),
            jax.ShapeDtypeStruct((N, NH), jnp.float32),
        ],
    )(xt, h, c, tx0, tx1, dinv, w0, w1a, w1b, wx, bc)


def kernel(x_seq, edge_index, W_i, b_i, Wch_i, bch_i, W_f, b_f, Wch_f, bch_f,
           W_c, b_c, Wch_c, bch_c, W_o, b_o, Wch_o, bch_o):
    src = edge_index[0]
    dst = edge_index[1]

    w0 = jnp.concatenate([Wch_i[0], Wch_f[0], Wch_c[0], Wch_o[0]], axis=1)
    w1 = jnp.concatenate([Wch_i[1], Wch_f[1], Wch_c[1], Wch_o[1]], axis=1)
    w1a = w1[0:NH]
    w1b = w1[NH:HID]
    wx = jnp.concatenate([W_i, W_f, W_c, W_o], axis=1)
    bc = jnp.concatenate([
        b_i + bch_i[None, :], b_f + bch_f[None, :],
        b_c + bch_c[None, :], b_o + bch_o[None, :]], axis=1)

    zeros_nh = jnp.zeros((N, NH), jnp.float32)
    ones_nh = jnp.ones((CH, NH), jnp.float32)

    srcd = jnp.concatenate(
        [src, jnp.zeros((DEG_EPAD - E,), jnp.int32)]).reshape(NC * NS * DEGC, CH)
    padc = jnp.concatenate([
        jnp.full((1, 1), float(DEG_EPAD - E), jnp.float32),
        jnp.zeros((N - 1, 1), jnp.float32)])

    dega, degb = _sc_degree(srcd, zeros_nh, ones_nh)
    h, cst = _tc_step0(x_seq[0], wx, bc)
    dinv, hs0, hs1 = _tc_prep(h, dega, degb, padc)

    sd = jnp.stack([src.reshape(NCHUNK, CH), dst.reshape(NCHUNK, CH)], axis=1)

    for t in range(1, 3):
        tx0, tx1 = _sc_segsum(hs0, hs1, sd, zeros_nh)
        h, cst, hs0, hs1 = _tc_step(
            x_seq[t], h, cst, tx0, tx1, dinv, w0, w1a, w1b, wx, bc)

    return h
